# Initial kernel scaffold; baseline (speedup 1.0000x reference)
import jax
import jax.numpy as jnp
from jax.experimental import pallas as pl


def kernel(x, m, f, W1, b1, W2, b2, W3, b3, W4, b4, WA, bA, WA1, bA1, edge_index):
    raise NotImplementedError("write your pallas kernel here")


# speedup vs baseline: 12.2763x; 12.2763x over previous
import functools

import jax
import jax.numpy as jnp
from jax import lax
from jax.experimental import pallas as pl
from jax.experimental.pallas import tpu as pltpu
from jax.experimental.pallas import tpu_sc as plsc

F32 = jnp.float32

NC = 2
NS = 16
NW = NC * NS
C = 128
ZR = 32


def _sc_scatter_add(n_acc, dp, e_pad):
    ew = e_pad // NW
    g_chunks = ew // C
    rows_per = n_acc // NS
    nb = 2 if dp >= 112 else 8
    t_outer = g_chunks // nb
    assert g_chunks % nb == 0
    mesh = plsc.VectorSubcoreMesh(core_axis_name="c", subcore_axis_name="s",
                                  num_cores=NC, num_subcores=NS)

    @functools.partial(
        pl.kernel,
        out_type=jax.ShapeDtypeStruct((NC, n_acc, dp), F32),
        mesh=mesh,
        scratch_types=[
            pltpu.VMEM((g_chunks, C), jnp.int32),
            pltpu.VMEM((g_chunks, C), jnp.int32),
            [pltpu.VMEM((C, dp), F32) for _ in range(nb)],
            pltpu.VMEM((ZR, dp), F32),
            pltpu.VMEM_SHARED((n_acc, dp), F32),
            [pltpu.SemaphoreType.DMA for _ in range(nb)],
            [pltpu.SemaphoreType.DMA for _ in range(nb)],
        ],
        compiler_params=pltpu.CompilerParams(use_tc_tiling_on_sc=False),
        interpret=False,
    )
    def k(y_hbm, src_hbm, dst_hbm, zrows_hbm, out_hbm,
          sidx, didx, rows, zbuf, acc, gsem, ssem):
        c = lax.axis_index("c")
        s = lax.axis_index("s")
        wid = c * NS + s

        pltpu.sync_copy(src_hbm.at[wid], sidx)
        pltpu.sync_copy(dst_hbm.at[wid], didx)

        pltpu.sync_copy(zrows_hbm, zbuf)
        r0 = s * rows_per
        for j in range(rows_per // ZR):
            pltpu.sync_copy(zbuf, acc.at[pl.ds(r0 + j * ZR, ZR)])
        plsc.subcore_barrier()

        def gather_start(g, b):
            pltpu.async_copy(y_hbm.at[sidx.at[g]], rows[b], gsem[b])

        def gather_wait(g, b):
            pltpu.make_async_copy(y_hbm.at[sidx.at[g]], rows[b],
                                  gsem[b]).wait()

        def scat_start(g, b):
            pltpu.async_copy(rows[b], acc.at[didx.at[g]], ssem[b], add=True)

        def scat_wait(g, b):
            pltpu.make_async_copy(rows[b], acc.at[didx.at[g]],
                                  ssem[b]).wait()

        for b in range(nb):
            gather_start(b, b)

        def outer(t, carry):
            base = t * nb
            for b in range(nb):
                gather_wait(base + b, b)
                scat_start(base + b, b)
            @pl.when(t + 1 < t_outer)
            def _():
                for b in range(nb):
                    scat_wait(base + b, b)
                    gather_start(base + nb + b, b)
            return carry

        lax.fori_loop(0, t_outer, outer, 0)
        for b in range(nb):
            scat_wait((t_outer - 1) * nb + b, b)

        plsc.subcore_barrier()
        pltpu.sync_copy(acc.at[pl.ds(r0, rows_per)],
                        out_hbm.at[c, pl.ds(r0, rows_per)])

    return k


def _sc_degree(n_acc, e_pad):
    dp = 16
    ew = e_pad // NW
    g_chunks = ew // C
    rows_per = n_acc // NS
    mesh = plsc.VectorSubcoreMesh(core_axis_name="c", subcore_axis_name="s",
                                  num_cores=NC, num_subcores=NS)

    @functools.partial(
        pl.kernel,
        out_type=jax.ShapeDtypeStruct((NC, n_acc, dp), F32),
        mesh=mesh,
        scratch_types=[
            pltpu.VMEM((g_chunks, C), jnp.int32),
            pltpu.VMEM((C, dp), F32),
            pltpu.VMEM((ZR, dp), F32),
            pltpu.VMEM_SHARED((n_acc, dp), F32),
            pltpu.SemaphoreType.DMA,
        ],
        compiler_params=pltpu.CompilerParams(use_tc_tiling_on_sc=False),
        interpret=False,
    )
    def k(dst_hbm, zrows_hbm, ones_hbm, out_hbm, didx, ones, zbuf, acc, sem):
        c = lax.axis_index("c")
        s = lax.axis_index("s")
        wid = c * NS + s

        pltpu.sync_copy(dst_hbm.at[wid], didx)
        pltpu.sync_copy(ones_hbm, ones)
        pltpu.sync_copy(zrows_hbm, zbuf)
        r0 = s * rows_per
        for j in range(rows_per // ZR):
            pltpu.sync_copy(zbuf, acc.at[pl.ds(r0 + j * ZR, ZR)])
        plsc.subcore_barrier()

        def body(g, carry):
            pltpu.async_copy(ones, acc.at[didx.at[g]], sem, add=True)
            return carry

        lax.fori_loop(0, g_chunks, body, 0)

        def drain(g, carry):
            pltpu.make_async_copy(ones, acc.at[didx.at[g]], sem).wait()
            return carry

        lax.fori_loop(0, g_chunks, drain, 0)

        plsc.subcore_barrier()
        pltpu.sync_copy(acc.at[pl.ds(r0, rows_per)],
                        out_hbm.at[c, pl.ds(r0, rows_per)])

    return k


def _tc_first(n, n_acc, d_in, dp_out, blk):
    def body(x_ref, w_ref, degp_ref, dis_ref, z_ref):
        deg = degp_ref[0] + degp_ref[1] + 1.0
        dis = lax.rsqrt(deg)
        dis_ref[...] = dis
        z_ref[...] = dis[:, 0:1] * jnp.dot(
            x_ref[...], w_ref[...], preferred_element_type=F32)

    return pl.pallas_call(
        body,
        grid=(n // blk,),
        in_specs=[
            pl.BlockSpec((blk, d_in), lambda i: (i, 0)),
            pl.BlockSpec((d_in, dp_out), lambda i: (0, 0)),
            pl.BlockSpec((NC, blk, 16), lambda i: (0, i, 0)),
        ],
        out_specs=[
            pl.BlockSpec((blk, 16), lambda i: (i, 0)),
            pl.BlockSpec((blk, dp_out), lambda i: (i, 0)),
        ],
        out_shape=[
            jax.ShapeDtypeStruct((n, 16), F32),
            jax.ShapeDtypeStruct((n, dp_out), F32),
        ],
        interpret=False,
    )


def _tc_mid(n, n_acc, dp_in, dp_out, blk):
    def body(aggp_ref, z_ref, dis_ref, b_ref, w_ref, zn_ref):
        dcol = dis_ref[:, 0:1]
        h = dcol * (aggp_ref[0] + aggp_ref[1] + z_ref[...]) + b_ref[...]
        h = jnp.maximum(h, 0.0)
        zn_ref[...] = dcol * jnp.dot(
            h, w_ref[...], preferred_element_type=F32)

    return pl.pallas_call(
        body,
        grid=(n // blk,),
        in_specs=[
            pl.BlockSpec((NC, blk, dp_in), lambda i: (0, i, 0)),
            pl.BlockSpec((blk, dp_in), lambda i: (i, 0)),
            pl.BlockSpec((blk, 16), lambda i: (i, 0)),
            pl.BlockSpec((1, dp_in), lambda i: (0, 0)),
            pl.BlockSpec((dp_in, dp_out), lambda i: (0, 0)),
        ],
        out_specs=pl.BlockSpec((blk, dp_out), lambda i: (i, 0)),
        out_shape=jax.ShapeDtypeStruct((n, dp_out), F32),
        interpret=False,
    )


def _tc_last(n, n_acc, dp_in, blk):
    def body(aggp_ref, z_ref, dis_ref, b_ref, out_ref):
        dcol = dis_ref[:, 0:1]
        o = dcol * (aggp_ref[0] + aggp_ref[1] + z_ref[...]) + b_ref[...]
        out_ref[...] = o[:, 0:1]

    return pl.pallas_call(
        body,
        grid=(n // blk,),
        in_specs=[
            pl.BlockSpec((NC, blk, dp_in), lambda i: (0, i, 0)),
            pl.BlockSpec((blk, dp_in), lambda i: (i, 0)),
            pl.BlockSpec((blk, 16), lambda i: (i, 0)),
            pl.BlockSpec((1, dp_in), lambda i: (0, 0)),
        ],
        out_specs=pl.BlockSpec((blk, 1), lambda i: (i, 0)),
        out_shape=jax.ShapeDtypeStruct((n, 1), F32),
        interpret=False,
    )


def _pad2(a, r, c):
    return jnp.zeros((r, c), F32).at[:a.shape[0], :a.shape[1]].set(a)


def kernel(x, m, f, W1, b1, W2, b2, W3, b3, W4, b4, WA, bA, WA1, bA1,
           edge_index):
    n, d_in = x.shape
    e = edge_index.shape[1]
    dp = (112, 64, 32, 16)
    n_acc = ((n + 1 + 1023) // 1024) * 1024
    e_pad = ((e + NW * C * 8 - 1) // (NW * C * 8)) * (NW * C * 8)
    g_chunks = e_pad // (NW * C)
    blk = 1000 if n % 1000 == 0 else 8 * (n // 8)

    src_p = jnp.concatenate(
        [edge_index[0], jnp.zeros((e_pad - e,), jnp.int32)])
    dst_p = jnp.concatenate(
        [edge_index[1], jnp.full((e_pad - e,), n, jnp.int32)])
    src3 = src_p.reshape(NW, g_chunks, C)
    dst3 = dst_p.reshape(NW, g_chunks, C)

    w1p = _pad2(W1, d_in, dp[0])
    w2p = _pad2(W2, dp[0], dp[1])
    w3p = _pad2(W3, dp[1], dp[2])
    w4p = _pad2(W4, dp[2], dp[3])
    b1p = _pad2(b1[None, :], 1, dp[0])
    b2p = _pad2(b2[None, :], 1, dp[1])
    b3p = _pad2(b3[None, :], 1, dp[2])
    b4p = _pad2(b4[None, :], 1, dp[3])

    zrows16 = jnp.zeros((ZR, 16), F32)
    ones16 = jnp.ones((C, 16), F32)

    degp = _sc_degree(n_acc, e_pad)(dst3, zrows16, ones16)

    dis16, z1 = _tc_first(n, n_acc, d_in, dp[0], blk)(x, w1p, degp)

    agg1 = _sc_scatter_add(n_acc, dp[0], e_pad)(
        z1, src3, dst3, jnp.zeros((ZR, dp[0]), F32))
    z2 = _tc_mid(n, n_acc, dp[0], dp[1], blk)(agg1, z1, dis16, b1p, w2p)

    agg2 = _sc_scatter_add(n_acc, dp[1], e_pad)(
        z2, src3, dst3, jnp.zeros((ZR, dp[1]), F32))
    z3 = _tc_mid(n, n_acc, dp[1], dp[2], blk)(agg2, z2, dis16, b2p, w3p)

    agg3 = _sc_scatter_add(n_acc, dp[2], e_pad)(
        z3, src3, dst3, jnp.zeros((ZR, dp[2]), F32))
    z4 = _tc_mid(n, n_acc, dp[2], dp[3], blk)(agg3, z3, dis16, b3p, w4p)

    agg4 = _sc_scatter_add(n_acc, dp[3], e_pad)(
        z4, src3, dst3, jnp.zeros((ZR, dp[3]), F32))
    h = _tc_last(n, n_acc, dp[3], blk)(agg4, z4, dis16, b4p)
    return h
